# baseline (speedup 1.0000x reference)
import functools

import jax
import jax.numpy as jnp
from jax import lax
from jax.experimental import pallas as pl
from jax.experimental.pallas import tpu as pltpu
from jax.experimental.pallas import tpu_sc as plsc

N = 10000
D = 128
E = 320000
L = 4
BN_EPS = 1e-5

NC = 2
NS = 16
NW = NC * NS

G = 128
NBUF = 2
K = 8
E_TILE = 10240
STEPS = E_TILE // G
NWIN = STEPS // K
E_PAD = NW * E_TILE

H_PAD = 10240
ROWS_PER_TILE = H_PAD // NS
DUMMY_ROW = N


def _sc_agg_body(h_hbm, src_hbm, dstw_hbm, zeros_hbm, out_hbm,
                 src_v, dstw_v, rows_v, agg_sh, wsem, gsem, ssem):
    c = lax.axis_index("c")
    s = lax.axis_index("s")

    pltpu.sync_copy(src_hbm.at[c, s], src_v)
    pltpu.async_copy(dstw_hbm.at[c, s, 0], dstw_v.at[0], wsem.at[0])
    pltpu.async_copy(dstw_hbm.at[c, s, 1], dstw_v.at[1], wsem.at[1])

    pltpu.sync_copy(zeros_hbm,
                    agg_sh.at[pl.ds(s * ROWS_PER_TILE, ROWS_PER_TILE)])
    plsc.subcore_barrier()

    for b in range(NBUF):
        pltpu.async_copy(h_hbm.at[src_v.at[b]], rows_v.at[b], gsem.at[b])

    def wloop(wo, carry):
        for half in range(2):
            w = 2 * wo + half
            pltpu.make_async_copy(dstw_hbm.at[c, s, w], dstw_v.at[half],
                                  wsem.at[half]).wait()
            for k in range(K):
                b = k % NBUF
                j = w * K + k
                pltpu.make_async_copy(h_hbm.at[src_v.at[j]], rows_v.at[b],
                                      gsem.at[b]).wait()
                pltpu.async_copy(rows_v.at[b], agg_sh.at[dstw_v.at[half, k]],
                                 ssem.at[b], add=True)
                pltpu.make_async_copy(rows_v.at[b],
                                      agg_sh.at[dstw_v.at[half, k]],
                                      ssem.at[b]).wait()
                pltpu.async_copy(h_hbm.at[src_v.at[j + NBUF]], rows_v.at[b],
                                 gsem.at[b])
            @pl.when(w + 2 < NWIN)
            def _():
                pltpu.async_copy(dstw_hbm.at[c, s, w + 2], dstw_v.at[half],
                                 wsem.at[half])
        return carry

    lax.fori_loop(0, NWIN // 2, wloop, 0, unroll=False)
    for b in range(NBUF):
        pltpu.make_async_copy(h_hbm.at[src_v.at[STEPS + b]], rows_v.at[b],
                              gsem.at[b]).wait()
    plsc.subcore_barrier()

    rows = pl.ds(s * ROWS_PER_TILE, ROWS_PER_TILE)
    pltpu.sync_copy(agg_sh.at[rows], out_hbm.at[c, rows])


def _sc_agg(h, src_t, dstw_t, zeros_blk):
    mesh = plsc.VectorSubcoreMesh(core_axis_name="c", subcore_axis_name="s")
    kern = pl.kernel(
        _sc_agg_body,
        out_type=jax.ShapeDtypeStruct((NC, H_PAD, D), jnp.float32),
        mesh=mesh,
        scratch_types=[
            pltpu.VMEM((STEPS + NBUF, G), jnp.int32),
            pltpu.VMEM((2, K, G), jnp.int32),
            pltpu.VMEM((NBUF, G, D), jnp.float32),
            pltpu.VMEM_SHARED((H_PAD, D), jnp.float32),
            pltpu.SemaphoreType.DMA((2,)),
            pltpu.SemaphoreType.DMA((NBUF,)),
            pltpu.SemaphoreType.DMA((NBUF,)),
        ],
    )
    return kern(h, src_t, dstw_t, zeros_blk)


def _tc_mlp_body(h_ref, agg_ref, w1_ref, b1_ref, w2_ref, b2_ref, o_ref):
    z = h_ref[...] + agg_ref[0] + agg_ref[1]
    z = jnp.dot(z, w1_ref[...], preferred_element_type=jnp.float32) + b1_ref[...]
    z = jnp.maximum(z, 0.0)
    z = jnp.dot(z, w2_ref[...], preferred_element_type=jnp.float32) + b2_ref[...]
    o_ref[...] = jnp.maximum(z, 0.0)


def _tc_mlp(h, aggp, w1f, b1f, w2, b2):
    B = 1024
    grid = (H_PAD // B,)
    return pl.pallas_call(
        _tc_mlp_body,
        grid=grid,
        in_specs=[
            pl.BlockSpec((B, D), lambda i: (i, 0)),
            pl.BlockSpec((NC, B, D), lambda i: (0, i, 0)),
            pl.BlockSpec((D, D), lambda i: (0, 0)),
            pl.BlockSpec((1, D), lambda i: (0, 0)),
            pl.BlockSpec((D, D), lambda i: (0, 0)),
            pl.BlockSpec((1, D), lambda i: (0, 0)),
        ],
        out_specs=pl.BlockSpec((B, D), lambda i: (i, 0)),
        out_shape=jax.ShapeDtypeStruct((H_PAD, D), jnp.float32),
    )(h, aggp, w1f, b1f, w2, b2)


def _tc_out_body(h_ref, w_ref, b_ref, o_ref):
    o_ref[...] = (
        jnp.dot(h_ref[...], w_ref[...], preferred_element_type=jnp.float32)
        + b_ref[...]
    )


def _tc_out(h, w_out, b_out):
    B = 1024
    grid = (H_PAD // B,)
    return pl.pallas_call(
        _tc_out_body,
        grid=grid,
        in_specs=[
            pl.BlockSpec((B, D), lambda i: (i, 0)),
            pl.BlockSpec((D, D), lambda i: (0, 0)),
            pl.BlockSpec((1, D), lambda i: (0, 0)),
        ],
        out_specs=pl.BlockSpec((B, D), lambda i: (i, 0)),
        out_shape=jax.ShapeDtypeStruct((H_PAD, D), jnp.float32),
    )(h, w_out, b_out)


def kernel(x, edge_index, W1, b1, gamma, beta, running_mean, running_var,
           W2, b2, W_out, b_out):
    src = edge_index[0]
    dst = edge_index[1]

    pad = E_PAD - E
    src_p = jnp.concatenate([src, jnp.zeros((pad,), jnp.int32)])
    dst_p = jnp.concatenate([dst, jnp.full((pad,), DUMMY_ROW, jnp.int32)])
    src_t = jnp.concatenate(
        [src_p.reshape(NW, STEPS, G),
         jnp.zeros((NW, NBUF, G), jnp.int32)], axis=1
    ).reshape(NC, NS, STEPS + NBUF, G)
    dstw_t = dst_p.reshape(NC, NS, NWIN, K, G)

    scale = gamma * lax.rsqrt(running_var + BN_EPS)
    W1f = W1 * scale[:, None, :]
    b1f = (b1 - running_mean) * scale + beta

    h = jnp.pad(x, ((0, H_PAD - N), (0, 0)))
    zeros_blk = jnp.zeros((ROWS_PER_TILE, D), jnp.float32)

    for i in range(L):
        aggp = _sc_agg(h, src_t, dstw_t, zeros_blk)
        h = _tc_mlp(h, aggp, W1f[i], b1f[i][None, :], W2[i], b2[i][None, :])

    out = _tc_out(h, W_out, b_out[None, :])
    return out[:N]


# speedup vs baseline: 1.3950x; 1.3950x over previous
import functools

import jax
import jax.numpy as jnp
from jax import lax
from jax.experimental import pallas as pl
from jax.experimental.pallas import tpu as pltpu
from jax.experimental.pallas import tpu_sc as plsc

N = 10000
D = 128
E = 320000
L = 4
BN_EPS = 1e-5

NC = 2
NS = 16
NW = NC * NS

G = 128
STEPS_0 = 96
STEPS_1 = 64
SMAX = max(STEPS_0, STEPS_1)
E_PAD = NS * G * (STEPS_0 + STEPS_1)

H_PAD = 10240
ROWS_PER_TILE = H_PAD // NS
DUMMY_ROW = N


def _sc_agg_body(h_hbm, src_hbm, dst_hbm, zeros_hbm, out_hbm,
                 src_v, dst_v, rows_v, agg_sh, sem):
    c = lax.axis_index("c")
    s = lax.axis_index("s")

    pltpu.sync_copy(src_hbm.at[c, s], src_v)
    pltpu.sync_copy(dst_hbm.at[c, s], dst_v)

    pltpu.sync_copy(zeros_hbm,
                    agg_sh.at[pl.ds(s * ROWS_PER_TILE, ROWS_PER_TILE)])
    plsc.subcore_barrier()

    nsteps = jnp.where(c == 0, STEPS_0, STEPS_1)

    def step(j, carry):
        pltpu.async_copy(h_hbm.at[src_v.at[j]], rows_v, sem).wait()
        pltpu.sync_copy(rows_v, agg_sh.at[dst_v.at[j]], add=True)
        return carry

    lax.fori_loop(0, nsteps, step, 0, unroll=False)
    plsc.subcore_barrier()

    rows = pl.ds(s * ROWS_PER_TILE, ROWS_PER_TILE)
    pltpu.sync_copy(agg_sh.at[rows], out_hbm.at[c, rows])


def _sc_agg(h, src_t, dst_t, zeros_blk):
    mesh = plsc.VectorSubcoreMesh(core_axis_name="c", subcore_axis_name="s")
    kern = pl.kernel(
        _sc_agg_body,
        out_type=jax.ShapeDtypeStruct((NC, H_PAD, D), jnp.float32),
        mesh=mesh,
        scratch_types=[
            pltpu.VMEM((SMAX, G), jnp.int32),
            pltpu.VMEM((SMAX, G), jnp.int32),
            pltpu.VMEM((G, D), jnp.float32),
            pltpu.VMEM_SHARED((H_PAD, D), jnp.float32),
            pltpu.SemaphoreType.DMA,
        ],
    )
    return kern(h, src_t, dst_t, zeros_blk)


def _tc_mlp_body(h_ref, agg_ref, w1_ref, b1_ref, w2_ref, b2_ref, o_ref):
    z = h_ref[...] + agg_ref[0] + agg_ref[1]
    z = jnp.dot(z, w1_ref[...], preferred_element_type=jnp.float32) + b1_ref[...]
    z = jnp.maximum(z, 0.0)
    z = jnp.dot(z, w2_ref[...], preferred_element_type=jnp.float32) + b2_ref[...]
    o_ref[...] = jnp.maximum(z, 0.0)


def _tc_mlp(h, aggp, w1f, b1f, w2, b2):
    B = 1024
    grid = (H_PAD // B,)
    return pl.pallas_call(
        _tc_mlp_body,
        grid=grid,
        in_specs=[
            pl.BlockSpec((B, D), lambda i: (i, 0)),
            pl.BlockSpec((NC, B, D), lambda i: (0, i, 0)),
            pl.BlockSpec((D, D), lambda i: (0, 0)),
            pl.BlockSpec((1, D), lambda i: (0, 0)),
            pl.BlockSpec((D, D), lambda i: (0, 0)),
            pl.BlockSpec((1, D), lambda i: (0, 0)),
        ],
        out_specs=pl.BlockSpec((B, D), lambda i: (i, 0)),
        out_shape=jax.ShapeDtypeStruct((H_PAD, D), jnp.float32),
    )(h, aggp, w1f, b1f, w2, b2)


def _tc_out_body(h_ref, w_ref, b_ref, o_ref):
    o_ref[...] = (
        jnp.dot(h_ref[...], w_ref[...], preferred_element_type=jnp.float32)
        + b_ref[...]
    )


def _tc_out(h, w_out, b_out):
    B = 1024
    grid = (H_PAD // B,)
    return pl.pallas_call(
        _tc_out_body,
        grid=grid,
        in_specs=[
            pl.BlockSpec((B, D), lambda i: (i, 0)),
            pl.BlockSpec((D, D), lambda i: (0, 0)),
            pl.BlockSpec((1, D), lambda i: (0, 0)),
        ],
        out_specs=pl.BlockSpec((B, D), lambda i: (i, 0)),
        out_shape=jax.ShapeDtypeStruct((H_PAD, D), jnp.float32),
    )(h, w_out, b_out)


def kernel(x, edge_index, W1, b1, gamma, beta, running_mean, running_var,
           W2, b2, W_out, b_out):
    src = edge_index[0]
    dst = edge_index[1]

    pad = E_PAD - E
    src_p = jnp.concatenate([src, jnp.zeros((pad,), jnp.int32)])
    dst_p = jnp.concatenate([dst, jnp.full((pad,), DUMMY_ROW, jnp.int32)])
    e0 = NS * STEPS_0 * G

    def to_tiles(p):
        t0 = p[:e0].reshape(NS, STEPS_0, G)
        t0 = jnp.pad(t0, ((0, 0), (0, SMAX - STEPS_0), (0, 0)))
        t1 = p[e0:].reshape(NS, STEPS_1, G)
        t1 = jnp.pad(t1, ((0, 0), (0, SMAX - STEPS_1), (0, 0)))
        return jnp.stack([t0, t1])

    src_t = to_tiles(src_p)
    dst_t = to_tiles(dst_p)

    scale = gamma * lax.rsqrt(running_var + BN_EPS)
    W1f = W1 * scale[:, None, :]
    b1f = (b1 - running_mean) * scale + beta

    h = jnp.pad(x, ((0, H_PAD - N), (0, 0)))
    zeros_blk = jnp.zeros((ROWS_PER_TILE, D), jnp.float32)

    for i in range(L):
        aggp = _sc_agg(h, src_t, dst_t, zeros_blk)
        h = _tc_mlp(h, aggp, W1f[i], b1f[i][None, :], W2[i], b2[i][None, :])

    out = _tc_out(h, W_out, b_out[None, :])
    return out[:N]
